# Initial kernel scaffold; baseline (speedup 1.0000x reference)
import jax
import jax.numpy as jnp
from jax.experimental import pallas as pl


def kernel(X, edge_weight_param, W1, b1, Wfc, bfc):
    raise NotImplementedError("write your pallas kernel here")


# speedup vs baseline: 925.0783x; 925.0783x over previous
import jax
import jax.numpy as jnp
from jax.experimental import pallas as pl
from jax.experimental.pallas import tpu as pltpu

N = 62
B = 256
F_IN = 32
H = 64
C = 3


def _prop_kernel(xp_ref, p_ref, x2_ref, wd_ref):
    wd_ref[...] = jnp.zeros((64, 64), jnp.float32)
    for i in range(N):
        off = i * (i + 1) // 2
        wd_ref[i, pl.ds(0, i + 1)] = p_ref[pl.ds(off, i + 1)]
    L = wd_ref[...]

    ri = jax.lax.broadcasted_iota(jnp.int32, (64, 64), 0)
    ci = jax.lax.broadcasted_iota(jnp.int32, (64, 64), 1)
    eye = jnp.where((ri == ci) & (ri < N), 1.0, 0.0).astype(jnp.float32)

    Wd = L + L.T - L * eye
    A = Wd + eye
    deg = jnp.sum(jnp.abs(Wd), axis=1, keepdims=True) + 1.0
    dis = jax.lax.rsqrt(deg)
    S = dis * A * dis.reshape(1, 64)
    S2 = jnp.dot(S, S, preferred_element_type=jnp.float32)

    x2_ref[...] = jnp.dot(S2[:N, :N], xp_ref[...],
                          preferred_element_type=jnp.float32)


def _head_kernel(x2_ref, w1_ref, b1_ref, wfc_ref, bfc_ref, out_ref):
    T = jax.lax.dot_general(x2_ref[...], w1_ref[...], (((1,), (1,)), ((), ())),
                            preferred_element_type=jnp.float32)
    T = jnp.maximum(T + b1_ref[...], 0.0)

    pooled = jnp.sum(T.reshape(N, B, H), axis=0)

    logits = jax.lax.dot_general(pooled, wfc_ref[...], (((1,), (1,)), ((), ())),
                                 preferred_element_type=jnp.float32)
    logits = logits + bfc_ref[...]
    m = jnp.max(logits, axis=-1, keepdims=True)
    y = logits - m
    out_ref[...] = y - jnp.log(jnp.sum(jnp.exp(y), axis=-1, keepdims=True))


def kernel(X, edge_weight_param, W1, b1, Wfc, bfc):
    Xp = X.transpose(1, 0, 2).reshape(N, B * F_IN)
    X2 = pl.pallas_call(
        _prop_kernel,
        out_shape=jax.ShapeDtypeStruct((N, B * F_IN), jnp.float32),
        scratch_shapes=[pltpu.VMEM((64, 64), jnp.float32)],
    )(Xp, edge_weight_param)
    X2r = X2.reshape(N * B, F_IN)
    return pl.pallas_call(
        _head_kernel,
        out_shape=jax.ShapeDtypeStruct((B, C), jnp.float32),
    )(X2r, W1, b1.reshape(1, H), Wfc, bfc.reshape(1, C))


# speedup vs baseline: 934.5094x; 1.0102x over previous
import functools

import jax
import jax.numpy as jnp
from jax.experimental import pallas as pl
from jax.experimental.pallas import tpu as pltpu

N = 62
B = 256
F_IN = 32
H = 64
C = 3

BLOCK_B = 32
GRID = B // BLOCK_B


def _fused_kernel(x_ref, p_ref, w1_ref, b1_ref, wfc_ref, bfc_ref, out_ref,
                  s2_ref, w1g_ref, wfcg_ref):
    @pl.when(pl.program_id(0) == 0)
    def _init():
        s2_ref[...] = jnp.zeros((64, 64), jnp.float32)
        for i in range(N):
            off = i * (i + 1) // 2
            s2_ref[i, pl.ds(0, i + 1)] = p_ref[pl.ds(off, i + 1)]
        L = s2_ref[...]

        ri = jax.lax.broadcasted_iota(jnp.int32, (64, 64), 0)
        ci = jax.lax.broadcasted_iota(jnp.int32, (64, 64), 1)
        eye = jnp.where((ri == ci) & (ri < N), 1.0, 0.0).astype(jnp.float32)

        Wd = L + L.T - L * eye
        A = Wd + eye
        deg = jnp.sum(jnp.abs(Wd), axis=1, keepdims=True) + 1.0
        dis = jax.lax.rsqrt(deg)
        S = dis * A * dis.reshape(1, 64)
        s2_ref[...] = jnp.dot(S, S, preferred_element_type=jnp.float32)

        w1 = w1_ref[...]
        w1g_ref[...] = jnp.zeros((2 * H, 2 * F_IN), jnp.float32)
        w1g_ref[0:H, 0:F_IN] = w1
        w1g_ref[H:2 * H, F_IN:2 * F_IN] = w1
        wfc = wfc_ref[...]
        wfcg_ref[...] = jnp.zeros((2 * H, 2 * H), jnp.float32)
        wfcg_ref[0:C, 0:H] = wfc
        wfcg_ref[H:H + C, H:2 * H] = wfc

    s2 = s2_ref[0:N, 0:N]
    w1g = w1g_ref[...]
    wfcg = wfcg_ref[...]
    b1 = b1_ref[...]
    b1g = jnp.concatenate([b1, b1], axis=1)
    bfc = bfc_ref[...]

    cols = [None] * BLOCK_B
    for i in range(BLOCK_B):
        cols[i] = x_ref[i]
    xchunk = jnp.concatenate(cols, axis=1)
    x2chunk = jnp.dot(s2, xchunk, preferred_element_type=jnp.float32)

    for i in range(BLOCK_B // 2):
        x2 = x2chunk[:, 2 * F_IN * i:2 * F_IN * (i + 1)]
        t = jax.lax.dot_general(x2, w1g, (((1,), (1,)), ((), ())),
                                preferred_element_type=jnp.float32)
        t = jnp.maximum(t + b1g, 0.0)
        pooled = jnp.sum(t, axis=0, keepdims=True)
        logits = jax.lax.dot_general(pooled, wfcg, (((1,), (1,)), ((), ())),
                                     preferred_element_type=jnp.float32)
        for half in range(2):
            l = logits[:, H * half:H * half + C] + bfc
            m = jnp.max(l, axis=-1, keepdims=True)
            y = l - m
            out_ref[pl.ds(2 * i + half, 1), :] = (
                y - jnp.log(jnp.sum(jnp.exp(y), axis=-1, keepdims=True)))


def kernel(X, edge_weight_param, W1, b1, Wfc, bfc):
    return pl.pallas_call(
        _fused_kernel,
        grid=(GRID,),
        in_specs=[
            pl.BlockSpec((BLOCK_B, N, F_IN), lambda j: (j, 0, 0)),
            pl.BlockSpec((1953,), lambda j: (0,)),
            pl.BlockSpec((H, F_IN), lambda j: (0, 0)),
            pl.BlockSpec((1, H), lambda j: (0, 0)),
            pl.BlockSpec((C, H), lambda j: (0, 0)),
            pl.BlockSpec((1, C), lambda j: (0, 0)),
        ],
        out_specs=pl.BlockSpec((BLOCK_B, C), lambda j: (j, 0)),
        out_shape=jax.ShapeDtypeStruct((B, C), jnp.float32),
        scratch_shapes=[
            pltpu.VMEM((64, 64), jnp.float32),
            pltpu.VMEM((2 * H, 2 * F_IN), jnp.float32),
            pltpu.VMEM((2 * H, 2 * H), jnp.float32),
        ],
    )(X, edge_weight_param, W1, b1.reshape(1, H), Wfc, bfc.reshape(1, C))


# speedup vs baseline: 1128.6302x; 1.2077x over previous
import jax
import jax.numpy as jnp
from jax.experimental import pallas as pl
from jax.experimental.pallas import tpu as pltpu

N = 62
B = 256
F_IN = 32
H = 64
C = 3

BLOCK_B = 32
G = 4
GRID = B // BLOCK_B


def _fused_kernel(x_ref, p_ref, w1_ref, b1_ref, wfc_ref, bfc_ref, out_ref,
                  s2_ref, w1g_ref, pool_ref):
    j = pl.program_id(0)

    @pl.when(j == 0)
    def _init():
        s2_ref[...] = jnp.zeros((64, 64), jnp.float32)
        for i in range(N):
            off = i * (i + 1) // 2
            s2_ref[i, pl.ds(0, i + 1)] = p_ref[pl.ds(off, i + 1)]
        L = s2_ref[...]

        ri = jax.lax.broadcasted_iota(jnp.int32, (64, 64), 0)
        ci = jax.lax.broadcasted_iota(jnp.int32, (64, 64), 1)
        eye = jnp.where((ri == ci) & (ri < N), 1.0, 0.0).astype(jnp.float32)

        Wd = L + L.T - L * eye
        A = Wd + eye
        deg = jnp.sum(jnp.abs(Wd), axis=1, keepdims=True) + 1.0
        dis = jax.lax.rsqrt(deg)
        S = dis * A * dis.reshape(1, 64)
        s2_ref[...] = jnp.dot(S, S, preferred_element_type=jnp.float32)

        w1 = w1_ref[...]
        w1g_ref[...] = jnp.zeros((G * H, G * F_IN), jnp.float32)
        for g in range(G):
            w1g_ref[g * H:(g + 1) * H, g * F_IN:(g + 1) * F_IN] = w1

    s2 = s2_ref[0:N, 0:N]
    w1g = w1g_ref[...]
    b1 = b1_ref[...]
    b1g = jnp.concatenate([b1] * G, axis=1)

    xchunk = jnp.concatenate([x_ref[i] for i in range(BLOCK_B)], axis=1)
    x2chunk = jnp.dot(s2, xchunk, preferred_element_type=jnp.float32)

    for i in range(BLOCK_B // G):
        x2 = x2chunk[:, G * F_IN * i:G * F_IN * (i + 1)]
        t = jax.lax.dot_general(x2, w1g, (((1,), (1,)), ((), ())),
                                preferred_element_type=jnp.float32)
        t = jnp.maximum(t + b1g, 0.0)
        pooled = jnp.sum(t, axis=0, keepdims=True)
        for g in range(G):
            row = j * BLOCK_B + G * i + g
            pool_ref[pl.ds(row, 1), :] = pooled[:, g * H:(g + 1) * H]

    @pl.when(j == GRID - 1)
    def _head():
        pooled_all = pool_ref[...]
        logits = jax.lax.dot_general(pooled_all, wfc_ref[...],
                                     (((1,), (1,)), ((), ())),
                                     preferred_element_type=jnp.float32)
        logits = logits + bfc_ref[...]
        m = jnp.max(logits, axis=-1, keepdims=True)
        y = logits - m
        out_ref[...] = y - jnp.log(jnp.sum(jnp.exp(y), axis=-1, keepdims=True))


def kernel(X, edge_weight_param, W1, b1, Wfc, bfc):
    return pl.pallas_call(
        _fused_kernel,
        grid=(GRID,),
        in_specs=[
            pl.BlockSpec((BLOCK_B, N, F_IN), lambda j: (j, 0, 0)),
            pl.BlockSpec((1953,), lambda j: (0,)),
            pl.BlockSpec((H, F_IN), lambda j: (0, 0)),
            pl.BlockSpec((1, H), lambda j: (0, 0)),
            pl.BlockSpec((C, H), lambda j: (0, 0)),
            pl.BlockSpec((1, C), lambda j: (0, 0)),
        ],
        out_specs=pl.BlockSpec((B, C), lambda j: (0, 0)),
        out_shape=jax.ShapeDtypeStruct((B, C), jnp.float32),
        scratch_shapes=[
            pltpu.VMEM((64, 64), jnp.float32),
            pltpu.VMEM((G * H, G * F_IN), jnp.float32),
            pltpu.VMEM((B, H), jnp.float32),
        ],
    )(X, edge_weight_param, W1, b1.reshape(1, H), Wfc, bfc.reshape(1, C))
